# baseline (speedup 1.0000x reference)
import functools

import jax
import jax.numpy as jnp
from jax import lax
from jax.experimental import pallas as pl
from jax.experimental.pallas import tpu as pltpu
from jax.experimental.pallas import tpu_sc as plsc

FDIM = 32
H = 1024
W = 1024
HG = 2048
WG = 1024
NQ = HG * WG
HW = H * W

NCORES = 2
NSUB = 16
NWORK = NCORES * NSUB
QPW = NQ // NWORK
CHUNK = 256
NCHUNK = QPW // CHUNK
LANES = 16
NBUF = 2


def _transpose_body(fm_ref, t_ref):
    t_ref[...] = fm_ref[...].T


def _build_table(fm2):
    blk = 4096
    return pl.pallas_call(
        _transpose_body,
        grid=(HW // blk,),
        in_specs=[pl.BlockSpec((FDIM, blk), lambda i: (0, i))],
        out_specs=pl.BlockSpec((blk, FDIM), lambda i: (i, 0)),
        out_shape=jax.ShapeDtypeStruct((HW, FDIM), jnp.float32),
    )(fm2)


_MESH = plsc.VectorSubcoreMesh(
    core_axis_name="c", subcore_axis_name="s",
    num_cores=NCORES, num_subcores=NSUB,
)


@functools.partial(
    pl.kernel,
    mesh=_MESH,
    compiler_params=pltpu.CompilerParams(
        needs_layout_passes=False, use_tc_tiling_on_sc=False),
    out_type=jax.ShapeDtypeStruct((NQ, FDIM), jnp.float32),
    scratch_types=[
        pltpu.VMEM((NBUF, 2 * CHUNK), jnp.float32),
        pltpu.VMEM((NBUF, CHUNK), jnp.int32),
        pltpu.VMEM((NBUF, CHUNK), jnp.int32),
        pltpu.VMEM((NBUF, CHUNK), jnp.int32),
        pltpu.VMEM((NBUF, CHUNK), jnp.int32),
        pltpu.VMEM((NBUF, CHUNK), jnp.float32),
        pltpu.VMEM((NBUF, CHUNK), jnp.float32),
        pltpu.VMEM((NBUF, CHUNK, FDIM), jnp.float32),
        pltpu.VMEM((NBUF, CHUNK, FDIM), jnp.float32),
        pltpu.VMEM((NBUF, CHUNK, FDIM), jnp.float32),
        pltpu.VMEM((NBUF, CHUNK, FDIM), jnp.float32),
        pltpu.VMEM((NBUF, CHUNK, FDIM), jnp.float32),
        pltpu.SemaphoreType.DMA,
        pltpu.SemaphoreType.DMA,
        pltpu.SemaphoreType.DMA,
        pltpu.SemaphoreType.DMA,
        pltpu.SemaphoreType.DMA,
        pltpu.SemaphoreType.DMA,
    ],
)
def _sc_sample(coords, table, out,
               cbuf, i00v, i01v, i10v, i11v, wxv, wyv,
               v00v, v01v, v10v, v11v, outv,
               semi0, semi1, semg0, semg1, semo0, semo1):
    wid = lax.axis_index("c") * NSUB + lax.axis_index("s")
    qbase0 = wid * QPW
    iota = lax.iota(jnp.int32, LANES)
    semi = (semi0, semi1)
    semg = (semg0, semg1)
    semo = (semo0, semo1)
    idx_bufs = (i00v, i01v, i10v, i11v)
    v_bufs = (v00v, v01v, v10v, v11v)

    def fire_in(ci, b):
        src = coords.at[pl.ds((qbase0 + ci * CHUNK) * 2, 2 * CHUNK)]
        pltpu.async_copy(src, cbuf.at[b], semi[b])

    def wait_in(b):
        src = coords.at[pl.ds(0, 2 * CHUNK)]
        pltpu.make_async_copy(src, cbuf.at[b], semi[b]).wait()

    def prep(b):
        @pl.loop(0, CHUNK // LANES)
        def _g(g):
            base2 = (iota + g * LANES) * 2
            xg = plsc.load_gather(cbuf.at[b], [base2])
            yg = plsc.load_gather(cbuf.at[b], [base2 + 1])
            x = jnp.clip((xg + 1.0) * (0.5 * (W - 1)), 0.0, float(W - 1))
            y = jnp.clip((yg + 1.0) * (0.5 * (H - 1)), 0.0, float(H - 1))
            x0i = x.astype(jnp.int32)
            y0i = y.astype(jnp.int32)
            wx = x - x0i.astype(jnp.float32)
            wy = y - y0i.astype(jnp.float32)
            x1i = jnp.minimum(x0i + 1, W - 1)
            y1i = jnp.minimum(y0i + 1, H - 1)
            r0 = y0i * W
            r1 = y1i * W
            sl = pl.ds(g * LANES, LANES)
            i00v[b, sl] = r0 + x0i
            i01v[b, sl] = r0 + x1i
            i10v[b, sl] = r1 + x0i
            i11v[b, sl] = r1 + x1i
            wxv[b, sl] = wx
            wyv[b, sl] = wy

    def gather_descs(b):
        descs = []
        for iv, vv in zip(idx_bufs, v_bufs):
            for half in range(2):
                sl = pl.ds(half * 128, 128)
                descs.append((table.at[iv.at[b, sl]], vv.at[b].at[sl]))
        return descs

    def fire_gathers(b):
        for src, dst in gather_descs(b):
            pltpu.async_copy(src, dst, semg[b])

    def wait_gathers(b):
        for src, dst in gather_descs(b):
            pltpu.make_async_copy(src, dst, semg[b]).wait()

    def blend(b):
        @pl.loop(0, CHUNK // LANES)
        def _g(g):
            sl = pl.ds(g * LANES, LANES)
            wx = wxv[b, sl]
            wy = wyv[b, sl]
            u = 1.0 - wx
            v = 1.0 - wy
            wa = u * v
            wb = wx * v
            wc = u * wy
            wd = wx * wy
            row = iota + g * LANES
            for chb in range(0, FDIM, 16):
                accs = []
                for ch in range(chb, chb + 16):
                    col = jnp.full((LANES,), ch, jnp.int32)
                    acc = plsc.load_gather(v00v.at[b], [row, col]) * wa
                    acc = acc + plsc.load_gather(v01v.at[b], [row, col]) * wb
                    acc = acc + plsc.load_gather(v10v.at[b], [row, col]) * wc
                    acc = acc + plsc.load_gather(v11v.at[b], [row, col]) * wd
                    accs.append(acc)
                for k, ch in enumerate(range(chb, chb + 16)):
                    col = jnp.full((LANES,), ch, jnp.int32)
                    plsc.store_scatter(outv.at[b], [row, col], accs[k])

    def fire_out(ci, b):
        dst = out.at[pl.ds(qbase0 + ci * CHUNK, CHUNK)]
        pltpu.async_copy(outv.at[b], dst, semo[b])

    def wait_out(b):
        dst = out.at[pl.ds(0, CHUNK)]
        pltpu.make_async_copy(outv.at[b], dst, semo[b]).wait()

    fire_in(0, 0)
    wait_in(0)
    prep(0)
    fire_gathers(0)
    fire_in(1, 1)

    @pl.loop(0, NCHUNK, step=NBUF)
    def _main(base):
        for b in range(NBUF):
            ci = base + b
            nb = 1 - b
            wait_gathers(b)

            @pl.when(ci + 1 < NCHUNK)
            def _():
                wait_in(nb)
                prep(nb)
                fire_gathers(nb)

            @pl.when(ci >= NBUF)
            def _():
                wait_out(b)

            blend(b)
            fire_out(ci, b)

            @pl.when(ci + NBUF < NCHUNK)
            def _():
                fire_in(ci + NBUF, b)

    wait_out(0)
    wait_out(1)


def kernel(x_coords, fm):
    table = _build_table(fm.reshape(FDIM, HW))
    out = _sc_sample(x_coords.reshape(NQ * 2), table)
    return out.reshape(HG, WG, FDIM)


# speedup vs baseline: 1.8511x; 1.8511x over previous
import functools

import jax
import jax.numpy as jnp
from jax import lax
from jax.experimental import pallas as pl
from jax.experimental.pallas import tpu as pltpu
from jax.experimental.pallas import tpu_sc as plsc

FDIM = 32
H = 1024
W = 1024
HG = 2048
WG = 1024
NQ = HG * WG
HW = H * W

NCORES = 2
NSUB = 16
NWORK = NCORES * NSUB
QPW = NQ // NWORK
CHUNK = 256
NCHUNK = QPW // CHUNK
LANES = 16
NBUF = 2


def _transpose_body(fm_ref, t_ref):
    t_ref[...] = fm_ref[...].T


def _build_table(fm2):
    blk = 4096
    return pl.pallas_call(
        _transpose_body,
        grid=(HW // blk,),
        in_specs=[pl.BlockSpec((FDIM, blk), lambda i: (0, i))],
        out_specs=pl.BlockSpec((blk, FDIM), lambda i: (i, 0)),
        out_shape=jax.ShapeDtypeStruct((HW, FDIM), jnp.float32),
    )(fm2)


_MESH = plsc.VectorSubcoreMesh(
    core_axis_name="c", subcore_axis_name="s",
    num_cores=NCORES, num_subcores=NSUB,
)


@functools.partial(
    pl.kernel,
    mesh=_MESH,
    compiler_params=pltpu.CompilerParams(
        needs_layout_passes=False, use_tc_tiling_on_sc=False),
    out_type=jax.ShapeDtypeStruct((NQ, FDIM), jnp.float32),
    scratch_types=[
        pltpu.VMEM((NBUF, 2 * CHUNK), jnp.float32),
        pltpu.VMEM((NBUF, CHUNK), jnp.int32),
        pltpu.VMEM((NBUF, CHUNK), jnp.int32),
        pltpu.VMEM((NBUF, CHUNK), jnp.int32),
        pltpu.VMEM((NBUF, CHUNK), jnp.int32),
        pltpu.VMEM((NBUF, CHUNK), jnp.float32),
        pltpu.VMEM((NBUF, CHUNK), jnp.float32),
        pltpu.VMEM((NBUF, CHUNK, FDIM), jnp.float32),
        pltpu.VMEM((NBUF, CHUNK, FDIM), jnp.float32),
        pltpu.VMEM((NBUF, CHUNK, FDIM), jnp.float32),
        pltpu.VMEM((NBUF, CHUNK, FDIM), jnp.float32),
        pltpu.VMEM((NBUF, CHUNK, FDIM), jnp.float32),
        pltpu.SemaphoreType.DMA,
        pltpu.SemaphoreType.DMA,
        pltpu.SemaphoreType.DMA,
        pltpu.SemaphoreType.DMA,
        pltpu.SemaphoreType.DMA,
        pltpu.SemaphoreType.DMA,
    ],
)
def _sc_sample(coords, table, out,
               cbuf, i00v, i01v, i10v, i11v, wxv, wyv,
               v00v, v01v, v10v, v11v, outv,
               semi0, semi1, semg0, semg1, semo0, semo1):
    wid = lax.axis_index("c") * NSUB + lax.axis_index("s")
    qbase0 = wid * QPW
    iota = lax.iota(jnp.int32, LANES)
    semi = (semi0, semi1)
    semg = (semg0, semg1)
    semo = (semo0, semo1)
    idx_bufs = (i00v, i01v, i10v, i11v)
    v_bufs = (v00v, v01v, v10v, v11v)

    def fire_in(ci, b):
        src = coords.at[pl.ds((qbase0 + ci * CHUNK) * 2, 2 * CHUNK)]
        pltpu.async_copy(src, cbuf.at[b], semi[b])

    def wait_in(b):
        src = coords.at[pl.ds(0, 2 * CHUNK)]
        pltpu.make_async_copy(src, cbuf.at[b], semi[b]).wait()

    def prep(b):
        @pl.loop(0, CHUNK // LANES)
        def _g(g):
            base2 = (iota + g * LANES) * 2
            xg = plsc.load_gather(cbuf.at[b], [base2])
            yg = plsc.load_gather(cbuf.at[b], [base2 + 1])
            x = jnp.clip((xg + 1.0) * (0.5 * (W - 1)), 0.0, float(W - 1))
            y = jnp.clip((yg + 1.0) * (0.5 * (H - 1)), 0.0, float(H - 1))
            x0i = x.astype(jnp.int32)
            y0i = y.astype(jnp.int32)
            wx = x - x0i.astype(jnp.float32)
            wy = y - y0i.astype(jnp.float32)
            x1i = jnp.minimum(x0i + 1, W - 1)
            y1i = jnp.minimum(y0i + 1, H - 1)
            r0 = y0i * W
            r1 = y1i * W
            sl = pl.ds(g * LANES, LANES)
            i00v[b, sl] = r0 + x0i
            i01v[b, sl] = r0 + x1i
            i10v[b, sl] = r1 + x0i
            i11v[b, sl] = r1 + x1i
            wxv[b, sl] = wx
            wyv[b, sl] = wy

    def gather_descs(b):
        descs = []
        for iv, vv in zip(idx_bufs, v_bufs):
            for half in range(2):
                sl = pl.ds(half * 128, 128)
                descs.append((table.at[iv.at[b, sl]], vv.at[b].at[sl]))
        return descs

    def fire_gathers(b):
        for src, dst in gather_descs(b):
            pltpu.async_copy(src, dst, semg[b])

    def wait_gathers(b):
        for src, dst in gather_descs(b):
            pltpu.make_async_copy(src, dst, semg[b]).wait()

    def blend(b):
        @pl.loop(0, CHUNK // LANES)
        def _g(g):
            sl = pl.ds(g * LANES, LANES)
            wx = wxv[b, sl]
            wy = wyv[b, sl]
            u = 1.0 - wx
            v = 1.0 - wy
            wa = u * v
            wb = wx * v
            wc = u * wy
            wd = wx * wy
            row = iota + g * LANES
            for chb in range(0, FDIM, 16):
                accs = []
                for ch in range(chb, chb + 16):
                    col = (iota + ch) & (FDIM - 1)
                    acc = plsc.load_gather(v00v.at[b], [row, col]) * wa
                    acc = acc + plsc.load_gather(v01v.at[b], [row, col]) * wb
                    acc = acc + plsc.load_gather(v10v.at[b], [row, col]) * wc
                    acc = acc + plsc.load_gather(v11v.at[b], [row, col]) * wd
                    accs.append(acc)
                for k, ch in enumerate(range(chb, chb + 16)):
                    col = (iota + ch) & (FDIM - 1)
                    plsc.store_scatter(outv.at[b], [row, col], accs[k])

    def fire_out(ci, b):
        dst = out.at[pl.ds(qbase0 + ci * CHUNK, CHUNK)]
        pltpu.async_copy(outv.at[b], dst, semo[b])

    def wait_out(b):
        dst = out.at[pl.ds(0, CHUNK)]
        pltpu.make_async_copy(outv.at[b], dst, semo[b]).wait()

    fire_in(0, 0)
    wait_in(0)
    prep(0)
    fire_gathers(0)
    fire_in(1, 1)

    @pl.loop(0, NCHUNK, step=NBUF)
    def _main(base):
        for b in range(NBUF):
            ci = base + b
            nb = 1 - b
            wait_gathers(b)

            @pl.when(ci + 1 < NCHUNK)
            def _():
                wait_in(nb)
                prep(nb)
                fire_gathers(nb)

            @pl.when(ci >= NBUF)
            def _():
                wait_out(b)

            blend(b)
            fire_out(ci, b)

            @pl.when(ci + NBUF < NCHUNK)
            def _():
                fire_in(ci + NBUF, b)

    wait_out(0)
    wait_out(1)


def kernel(x_coords, fm):
    table = _build_table(fm.reshape(FDIM, HW))
    out = _sc_sample(x_coords.reshape(NQ * 2), table)
    return out.reshape(HG, WG, FDIM)
